# baseline (speedup 1.0000x reference)
import functools

import jax
import jax.numpy as jnp
from jax.experimental import pallas as pl

_CODE_DIM = 64
_K = 1024
_SOS_TOKEN = 1024
_N_ROWS = 65536
_BLOCK_ROWS = 1024


def _vq_argmin_kernel(x_ref, cbt_ref, out_ref):
    x = x_ref[...]
    cbt = cbt_ref[...]
    dot = jnp.dot(x, cbt, preferred_element_type=jnp.float32)
    x_sq = jnp.sum(x * x, axis=1, keepdims=True)
    cb_sq = jnp.sum(cbt * cbt, axis=0, keepdims=True)
    dists = (x_sq - 2.0 * dot) + cb_sq
    m = jnp.min(dists, axis=1, keepdims=True)
    iota = jax.lax.broadcasted_iota(jnp.int32, dists.shape, 1)
    idx = jnp.min(jnp.where(dists == m, iota, _K), axis=1)
    out_ref[...] = idx.astype(jnp.int32)


@functools.partial(jax.jit, static_argnames=("interpret",))
def _vq_indices(flat, cbt, interpret=False):
    return pl.pallas_call(
        _vq_argmin_kernel,
        grid=(_N_ROWS // _BLOCK_ROWS,),
        in_specs=[
            pl.BlockSpec((_BLOCK_ROWS, _CODE_DIM), lambda i: (i, 0)),
            pl.BlockSpec((_CODE_DIM, _K), lambda i: (0, 0)),
        ],
        out_specs=pl.BlockSpec((_BLOCK_ROWS,), lambda i: (i,)),
        out_shape=jax.ShapeDtypeStruct((_N_ROWS,), jnp.int32),
        interpret=interpret,
    )(flat, cbt)


def kernel(weights_dict, y, codebook):
    flat = weights_dict.reshape(_N_ROWS, _CODE_DIM)
    idx = _vq_indices(flat, codebook.T)
    sos = jnp.array([_SOS_TOKEN], dtype=jnp.int32)
    x_out = jnp.concatenate([sos, idx[:-1]])
    return (x_out, idx)


# speedup vs baseline: 1.6428x; 1.6428x over previous
import functools

import jax
import jax.numpy as jnp
from jax.experimental import pallas as pl

_CODE_DIM = 64
_K = 1024
_SOS_TOKEN = 1024
_N_ROWS = 65536
_BLOCK_ROWS = 1024


def _vq_argmin_kernel(x_ref, cbm2_ref, cbsq_ref, out_ref):
    x = x_ref[...]
    cbm2 = cbm2_ref[...]
    scores = jax.lax.dot_general(
        cbm2, x, (((1,), (1,)), ((), ())),
        preferred_element_type=jnp.float32)
    dists = scores + cbsq_ref[...]
    m = jnp.min(dists, axis=0, keepdims=True)
    iota = jax.lax.broadcasted_iota(jnp.int32, dists.shape, 0)
    idx = jnp.min(jnp.where(dists == m, iota, _K), axis=0)
    out_ref[...] = idx.astype(jnp.int32)


@functools.partial(jax.jit, static_argnames=("interpret",))
def _vq_indices(flat, cbm2, cbsq, interpret=False):
    return pl.pallas_call(
        _vq_argmin_kernel,
        grid=(_N_ROWS // _BLOCK_ROWS,),
        in_specs=[
            pl.BlockSpec((_BLOCK_ROWS, _CODE_DIM), lambda i: (i, 0)),
            pl.BlockSpec((_K, _CODE_DIM), lambda i: (0, 0)),
            pl.BlockSpec((_K, 1), lambda i: (0, 0)),
        ],
        out_specs=pl.BlockSpec((_BLOCK_ROWS,), lambda i: (i,)),
        out_shape=jax.ShapeDtypeStruct((_N_ROWS,), jnp.int32),
        interpret=interpret,
    )(flat, cbm2, cbsq)


def kernel(weights_dict, y, codebook):
    flat = weights_dict.reshape(_N_ROWS, _CODE_DIM)
    cbm2 = -2.0 * codebook
    cbsq = 0.25 * jnp.sum(cbm2 * cbm2, axis=1, keepdims=True)
    idx = _vq_indices(flat, cbm2, cbsq)
    sos = jnp.array([_SOS_TOKEN], dtype=jnp.int32)
    x_out = jnp.concatenate([sos, idx[:-1]])
    return (x_out, idx)


# speedup vs baseline: 1.6483x; 1.0033x over previous
import functools

import jax
import jax.numpy as jnp
from jax.experimental import pallas as pl
from jax.experimental.pallas import tpu as pltpu

_CODE_DIM = 64
_K = 1024
_SOS_TOKEN = 1024
_N_ROWS = 65536
_BLOCK_ROWS = 1024


def _vq_argmin_kernel(x_ref, cbm2_ref, cbsq_ref, out_ref):
    x = x_ref[...]
    cbm2 = cbm2_ref[...]
    scores = jax.lax.dot_general(
        cbm2, x, (((1,), (1,)), ((), ())),
        preferred_element_type=jnp.float32)
    dists = scores + cbsq_ref[...]
    m = jnp.min(dists, axis=0, keepdims=True)
    iota = jax.lax.broadcasted_iota(jnp.int32, dists.shape, 0)
    idx = jnp.min(jnp.where(dists == m, iota, _K), axis=0)
    out_ref[...] = idx.astype(jnp.int32)


@functools.partial(jax.jit, static_argnames=("interpret",))
def _vq_indices(flat, cbm2, cbsq, interpret=False):
    return pl.pallas_call(
        _vq_argmin_kernel,
        grid=(_N_ROWS // _BLOCK_ROWS,),
        in_specs=[
            pl.BlockSpec((_BLOCK_ROWS, _CODE_DIM), lambda i: (i, 0)),
            pl.BlockSpec((_K, _CODE_DIM), lambda i: (0, 0)),
            pl.BlockSpec((_K, 1), lambda i: (0, 0)),
        ],
        out_specs=pl.BlockSpec((_BLOCK_ROWS,), lambda i: (i,)),
        out_shape=jax.ShapeDtypeStruct((_N_ROWS,), jnp.int32),
        compiler_params=pltpu.CompilerParams(dimension_semantics=("parallel",)),
        interpret=interpret,
    )(flat, cbm2, cbsq)


def kernel(weights_dict, y, codebook):
    flat = weights_dict.reshape(_N_ROWS, _CODE_DIM)
    cbm2 = -2.0 * codebook
    cbsq = 0.25 * jnp.sum(cbm2 * cbm2, axis=1, keepdims=True)
    idx = _vq_indices(flat, cbm2, cbsq)
    sos = jnp.array([_SOS_TOKEN], dtype=jnp.int32)
    x_out = jnp.concatenate([sos, idx[:-1]])
    return (x_out, idx)


# speedup vs baseline: 2.1606x; 1.3108x over previous
import functools

import jax
import jax.numpy as jnp
from jax.experimental import pallas as pl
from jax.experimental.pallas import tpu as pltpu

_CODE_DIM = 64
_K = 1024
_SOS_TOKEN = 1024
_N_ROWS = 65536
_BLOCK_ROWS = 1024


def _vq_argmin_kernel(x_ref, cbm2_ref, cbsq_ref, out_ref):
    x = x_ref[...]
    cbm2 = cbm2_ref[...]
    scores = jax.lax.dot_general(
        cbm2, x, (((1,), (1,)), ((), ())),
        preferred_element_type=jnp.float32)
    dists = scores + cbsq_ref[...]
    idx = jnp.argmin(dists, axis=0)
    out_ref[...] = idx.astype(jnp.int32)


@functools.partial(jax.jit, static_argnames=("interpret",))
def _vq_indices(flat, cbm2, cbsq, interpret=False):
    return pl.pallas_call(
        _vq_argmin_kernel,
        grid=(_N_ROWS // _BLOCK_ROWS,),
        in_specs=[
            pl.BlockSpec((_BLOCK_ROWS, _CODE_DIM), lambda i: (i, 0)),
            pl.BlockSpec((_K, _CODE_DIM), lambda i: (0, 0)),
            pl.BlockSpec((_K, 1), lambda i: (0, 0)),
        ],
        out_specs=pl.BlockSpec((_BLOCK_ROWS,), lambda i: (i,)),
        out_shape=jax.ShapeDtypeStruct((_N_ROWS,), jnp.int32),
        compiler_params=pltpu.CompilerParams(dimension_semantics=("parallel",)),
        interpret=interpret,
    )(flat, cbm2, cbsq)


def kernel(weights_dict, y, codebook):
    flat = weights_dict.reshape(_N_ROWS, _CODE_DIM)
    cbm2 = -2.0 * codebook
    cbsq = 0.25 * jnp.sum(cbm2 * cbm2, axis=1, keepdims=True)
    idx = _vq_indices(flat, cbm2, cbsq)
    sos = jnp.array([_SOS_TOKEN], dtype=jnp.int32)
    x_out = jnp.concatenate([sos, idx[:-1]])
    return (x_out, idx)


# speedup vs baseline: 2.5721x; 1.1905x over previous
import functools

import jax
import jax.numpy as jnp
from jax.experimental import pallas as pl
from jax.experimental.pallas import tpu as pltpu

_CODE_DIM = 64
_K = 1024
_SOS_TOKEN = 1024
_N_ROWS = 65536
_BLOCK_ROWS = 2048


def _vq_argmin_kernel(x_ref, cbm2_ref, cbsq_ref, out_ref):
    x = x_ref[...]
    cbm2 = cbm2_ref[...]
    scores = jax.lax.dot_general(
        cbm2, x, (((1,), (1,)), ((), ())),
        preferred_element_type=jnp.float32)
    dists = scores + cbsq_ref[...]
    idx = jnp.argmin(dists, axis=0)
    out_ref[...] = idx.astype(jnp.int32)


@functools.partial(jax.jit, static_argnames=("interpret",))
def _vq_indices(flat, cbm2, cbsq, interpret=False):
    return pl.pallas_call(
        _vq_argmin_kernel,
        grid=(_N_ROWS // _BLOCK_ROWS,),
        in_specs=[
            pl.BlockSpec((_BLOCK_ROWS, _CODE_DIM), lambda i: (i, 0)),
            pl.BlockSpec((_K, _CODE_DIM), lambda i: (0, 0)),
            pl.BlockSpec((_K, 1), lambda i: (0, 0)),
        ],
        out_specs=pl.BlockSpec((_BLOCK_ROWS,), lambda i: (i,)),
        out_shape=jax.ShapeDtypeStruct((_N_ROWS,), jnp.int32),
        compiler_params=pltpu.CompilerParams(dimension_semantics=("parallel",)),
        interpret=interpret,
    )(flat, cbm2, cbsq)


def kernel(weights_dict, y, codebook):
    flat = weights_dict.reshape(_N_ROWS, _CODE_DIM)
    cbm2 = -2.0 * codebook
    cbsq = 0.25 * jnp.sum(cbm2 * cbm2, axis=1, keepdims=True)
    idx = _vq_indices(flat, cbm2, cbsq)
    sos = jnp.array([_SOS_TOKEN], dtype=jnp.int32)
    x_out = jnp.concatenate([sos, idx[:-1]])
    return (x_out, idx)


# speedup vs baseline: 2.7186x; 1.0569x over previous
import functools

import jax
import jax.numpy as jnp
from jax.experimental import pallas as pl
from jax.experimental.pallas import tpu as pltpu

_CODE_DIM = 64
_K = 1024
_SOS_TOKEN = 1024
_N_ROWS = 65536
_BLOCK_ROWS = 4096


def _vq_argmin_kernel(x_ref, cbm2_ref, cbsq_ref, out_ref):
    x = x_ref[...]
    cbm2 = cbm2_ref[...]
    scores = jax.lax.dot_general(
        cbm2, x, (((1,), (1,)), ((), ())),
        preferred_element_type=jnp.float32)
    dists = scores + cbsq_ref[...]
    idx = jnp.argmin(dists, axis=0)
    out_ref[...] = idx.astype(jnp.int32)


@functools.partial(jax.jit, static_argnames=("interpret",))
def _vq_indices(flat, cbm2, cbsq, interpret=False):
    return pl.pallas_call(
        _vq_argmin_kernel,
        grid=(_N_ROWS // _BLOCK_ROWS,),
        in_specs=[
            pl.BlockSpec((_BLOCK_ROWS, _CODE_DIM), lambda i: (i, 0)),
            pl.BlockSpec((_K, _CODE_DIM), lambda i: (0, 0)),
            pl.BlockSpec((_K, 1), lambda i: (0, 0)),
        ],
        out_specs=pl.BlockSpec((_BLOCK_ROWS,), lambda i: (i,)),
        out_shape=jax.ShapeDtypeStruct((_N_ROWS,), jnp.int32),
        compiler_params=pltpu.CompilerParams(dimension_semantics=("parallel",)),
        interpret=interpret,
    )(flat, cbm2, cbsq)


def kernel(weights_dict, y, codebook):
    flat = weights_dict.reshape(_N_ROWS, _CODE_DIM)
    cbm2 = -2.0 * codebook
    cbsq = 0.25 * jnp.sum(cbm2 * cbm2, axis=1, keepdims=True)
    idx = _vq_indices(flat, cbm2, cbsq)
    sos = jnp.array([_SOS_TOKEN], dtype=jnp.int32)
    x_out = jnp.concatenate([sos, idx[:-1]])
    return (x_out, idx)


# speedup vs baseline: 2.7212x; 1.0010x over previous
import functools

import jax
import jax.numpy as jnp
from jax.experimental import pallas as pl
from jax.experimental.pallas import tpu as pltpu

_CODE_DIM = 64
_K = 1024
_SOS_TOKEN = 1024
_N_ROWS = 65536
_BLOCK_ROWS = 8192


def _vq_argmin_kernel(x_ref, cbm2_ref, cbsq_ref, out_ref):
    x = x_ref[...]
    cbm2 = cbm2_ref[...]
    scores = jax.lax.dot_general(
        cbm2, x, (((1,), (1,)), ((), ())),
        preferred_element_type=jnp.float32)
    dists = scores + cbsq_ref[...]
    idx = jnp.argmin(dists, axis=0)
    out_ref[...] = idx.astype(jnp.int32)


@functools.partial(jax.jit, static_argnames=("interpret",))
def _vq_indices(flat, cbm2, cbsq, interpret=False):
    return pl.pallas_call(
        _vq_argmin_kernel,
        grid=(_N_ROWS // _BLOCK_ROWS,),
        in_specs=[
            pl.BlockSpec((_BLOCK_ROWS, _CODE_DIM), lambda i: (i, 0)),
            pl.BlockSpec((_K, _CODE_DIM), lambda i: (0, 0)),
            pl.BlockSpec((_K, 1), lambda i: (0, 0)),
        ],
        out_specs=pl.BlockSpec((_BLOCK_ROWS,), lambda i: (i,)),
        out_shape=jax.ShapeDtypeStruct((_N_ROWS,), jnp.int32),
        compiler_params=pltpu.CompilerParams(dimension_semantics=("parallel",)),
        interpret=interpret,
    )(flat, cbm2, cbsq)


def kernel(weights_dict, y, codebook):
    flat = weights_dict.reshape(_N_ROWS, _CODE_DIM)
    cbm2 = -2.0 * codebook
    cbsq = 0.25 * jnp.sum(cbm2 * cbm2, axis=1, keepdims=True)
    idx = _vq_indices(flat, cbm2, cbsq)
    sos = jnp.array([_SOS_TOKEN], dtype=jnp.int32)
    x_out = jnp.concatenate([sos, idx[:-1]])
    return (x_out, idx)


# speedup vs baseline: 2.7992x; 1.0287x over previous
import functools

import jax
import jax.numpy as jnp
from jax.experimental import pallas as pl
from jax.experimental.pallas import tpu as pltpu

_CODE_DIM = 64
_K = 1024
_SOS_TOKEN = 1024
_N_ROWS = 65536
_BLOCK_ROWS = 8192


def _vq_argmin_kernel(cb_ref, x_ref, xout_ref, yout_ref,
                      cbm2_scr, cbsq_scr, carry_scr):
    @pl.when(pl.program_id(0) == 0)
    def _init():
        cbm2 = -2.0 * cb_ref[...]
        cbm2_scr[...] = cbm2
        cbsq_scr[...] = 0.25 * jnp.sum(cbm2 * cbm2, axis=1, keepdims=True)
        carry_scr[0] = _SOS_TOKEN

    x = x_ref[...]
    scores = jax.lax.dot_general(
        cbm2_scr[...], x, (((1,), (1,)), ((), ())),
        preferred_element_type=jnp.float32)
    dists = scores + cbsq_scr[...]
    idx = jnp.argmin(dists, axis=0).astype(jnp.int32)
    yout_ref[...] = idx
    idx2 = idx.reshape(1, _BLOCK_ROWS)
    rolled = pltpu.roll(idx2, 1, 1)
    lane = jax.lax.broadcasted_iota(jnp.int32, (1, _BLOCK_ROWS), 1)
    xout_ref[...] = jnp.where(lane == 0, carry_scr[0], rolled).reshape(
        _BLOCK_ROWS)
    carry_scr[0] = yout_ref[_BLOCK_ROWS - 1]


@functools.partial(jax.jit, static_argnames=("interpret",))
def _vq_transform(flat, codebook, interpret=False):
    return pl.pallas_call(
        _vq_argmin_kernel,
        grid=(_N_ROWS // _BLOCK_ROWS,),
        in_specs=[
            pl.BlockSpec((_K, _CODE_DIM), lambda i: (0, 0)),
            pl.BlockSpec((_BLOCK_ROWS, _CODE_DIM), lambda i: (i, 0)),
        ],
        out_specs=[
            pl.BlockSpec((_BLOCK_ROWS,), lambda i: (i,)),
            pl.BlockSpec((_BLOCK_ROWS,), lambda i: (i,)),
        ],
        out_shape=[
            jax.ShapeDtypeStruct((_N_ROWS,), jnp.int32),
            jax.ShapeDtypeStruct((_N_ROWS,), jnp.int32),
        ],
        scratch_shapes=[
            pltpu.VMEM((_K, _CODE_DIM), jnp.float32),
            pltpu.VMEM((_K, 1), jnp.float32),
            pltpu.SMEM((1,), jnp.int32),
        ],
        interpret=interpret,
    )(codebook, flat)


def kernel(weights_dict, y, codebook):
    flat = weights_dict.reshape(_N_ROWS, _CODE_DIM)
    x_out, y_out = _vq_transform(flat, codebook)
    return (x_out, y_out)


# speedup vs baseline: 3.0620x; 1.0939x over previous
import functools

import jax
import jax.numpy as jnp
from jax.experimental import pallas as pl
from jax.experimental.pallas import tpu as pltpu

_CODE_DIM = 64
_K = 1024
_SOS_TOKEN = 1024
_N_TOK = 65536
_N_ROWS = _N_TOK // 2
_BLOCK_ROWS = 4096


def _vq_argmin_kernel(cb_ref, x_ref, ye_ref, yo_ref,
                      cbe_scr, cbo_scr, cbsq_scr):
    @pl.when(pl.program_id(0) == 0)
    def _init():
        cbm2 = -2.0 * cb_ref[...]
        zero = jnp.zeros_like(cbm2)
        cbe_scr[...] = jnp.concatenate([cbm2, zero], axis=1)
        cbo_scr[...] = jnp.concatenate([zero, cbm2], axis=1)
        cbsq_scr[...] = 0.25 * jnp.sum(cbm2 * cbm2, axis=1, keepdims=True)

    x2 = x_ref[...]
    dims = (((1,), (1,)), ((), ()))
    cbsq = cbsq_scr[...]
    se = jax.lax.dot_general(cbe_scr[...], x2, dims,
                             preferred_element_type=jnp.float32)
    so = jax.lax.dot_general(cbo_scr[...], x2, dims,
                             preferred_element_type=jnp.float32)
    ye_ref[...] = jnp.argmin(se + cbsq, axis=0).astype(jnp.int32)
    yo_ref[...] = jnp.argmin(so + cbsq, axis=0).astype(jnp.int32)


@functools.partial(jax.jit, static_argnames=("interpret",))
def _vq_indices(flat2, codebook, interpret=False):
    return pl.pallas_call(
        _vq_argmin_kernel,
        grid=(_N_ROWS // _BLOCK_ROWS,),
        in_specs=[
            pl.BlockSpec((_K, _CODE_DIM), lambda i: (0, 0)),
            pl.BlockSpec((_BLOCK_ROWS, 2 * _CODE_DIM), lambda i: (i, 0)),
        ],
        out_specs=[
            pl.BlockSpec((_BLOCK_ROWS,), lambda i: (i,)),
            pl.BlockSpec((_BLOCK_ROWS,), lambda i: (i,)),
        ],
        out_shape=[
            jax.ShapeDtypeStruct((_N_ROWS,), jnp.int32),
            jax.ShapeDtypeStruct((_N_ROWS,), jnp.int32),
        ],
        scratch_shapes=[
            pltpu.VMEM((_K, 2 * _CODE_DIM), jnp.float32),
            pltpu.VMEM((_K, 2 * _CODE_DIM), jnp.float32),
            pltpu.VMEM((_K, 1), jnp.float32),
        ],
        interpret=interpret,
    )(codebook, flat2)


def kernel(weights_dict, y, codebook):
    flat2 = weights_dict.reshape(_N_ROWS, 2 * _CODE_DIM)
    ye, yo = _vq_indices(flat2, codebook)
    idx = jnp.stack([ye, yo], axis=1).reshape(_N_TOK)
    sos = jnp.array([_SOS_TOKEN], dtype=jnp.int32)
    x_out = jnp.concatenate([sos, idx[:-1]])
    return (x_out, idx)


# speedup vs baseline: 3.7249x; 1.2165x over previous
import functools

import jax
import jax.numpy as jnp
from jax.experimental import pallas as pl
from jax.experimental.pallas import tpu as pltpu

_CODE_DIM = 64
_K = 1024
_SOS_TOKEN = 1024
_N_TOK = 65536
_N_ROWS = _N_TOK // 2
_BLOCK_ROWS = 4096
_BLOCK_TOK = 2 * _BLOCK_ROWS


def _riffle(e, o, n):
    c = jnp.concatenate([e, o], axis=1)
    iota = jax.lax.broadcasted_iota(jnp.int32, (1, 2 * n), 1)
    s = n
    while s >= 2:
        q = s // 2
        b = (iota // q) & 3
        cm = pltpu.roll(c, 2 * n - q, 1)
        cp = pltpu.roll(c, q, 1)
        c = jnp.where(b == 1, cm, jnp.where(b == 2, cp, c))
        s = q
    return c


def _vq_argmin_kernel(cb_ref, x_ref, xout_ref, yout_ref,
                      cbm2_scr, cbsq_scr, carry_scr):
    @pl.when(pl.program_id(0) == 0)
    def _init():
        cbm2 = -2.0 * cb_ref[...]
        cbm2_scr[...] = cbm2
        cbsq_scr[...] = 0.25 * jnp.sum(cbm2 * cbm2, axis=1, keepdims=True)
        carry_scr[0] = _SOS_TOKEN

    x2 = x_ref[...]
    xe = x2[:, :_CODE_DIM]
    xo = x2[:, _CODE_DIM:]
    dims = (((1,), (1,)), ((), ()))
    cbm2 = cbm2_scr[...]
    cbsq = cbsq_scr[...]
    se = jax.lax.dot_general(cbm2, xe, dims,
                             preferred_element_type=jnp.float32)
    so = jax.lax.dot_general(cbm2, xo, dims,
                             preferred_element_type=jnp.float32)
    idx_e = jnp.argmin(se + cbsq, axis=0).astype(jnp.int32)
    idx_o = jnp.argmin(so + cbsq, axis=0).astype(jnp.int32)
    y2 = _riffle(idx_e.reshape(1, _BLOCK_ROWS), idx_o.reshape(1, _BLOCK_ROWS),
                 _BLOCK_ROWS)
    yout_ref[...] = y2.reshape(_BLOCK_TOK)
    rolled = pltpu.roll(y2, 1, 1)
    lane = jax.lax.broadcasted_iota(jnp.int32, (1, _BLOCK_TOK), 1)
    xout_ref[...] = jnp.where(lane == 0, carry_scr[0], rolled).reshape(
        _BLOCK_TOK)
    carry_scr[0] = yout_ref[_BLOCK_TOK - 1]


@functools.partial(jax.jit, static_argnames=("interpret",))
def _vq_transform(flat2, codebook, interpret=False):
    return pl.pallas_call(
        _vq_argmin_kernel,
        grid=(_N_ROWS // _BLOCK_ROWS,),
        in_specs=[
            pl.BlockSpec((_K, _CODE_DIM), lambda i: (0, 0)),
            pl.BlockSpec((_BLOCK_ROWS, 2 * _CODE_DIM), lambda i: (i, 0)),
        ],
        out_specs=[
            pl.BlockSpec((_BLOCK_TOK,), lambda i: (i,)),
            pl.BlockSpec((_BLOCK_TOK,), lambda i: (i,)),
        ],
        out_shape=[
            jax.ShapeDtypeStruct((_N_TOK,), jnp.int32),
            jax.ShapeDtypeStruct((_N_TOK,), jnp.int32),
        ],
        scratch_shapes=[
            pltpu.VMEM((_K, _CODE_DIM), jnp.float32),
            pltpu.VMEM((_K, 1), jnp.float32),
            pltpu.SMEM((1,), jnp.int32),
        ],
        interpret=interpret,
    )(codebook, flat2)


def kernel(weights_dict, y, codebook):
    flat2 = weights_dict.reshape(_N_ROWS, 2 * _CODE_DIM)
    x_out, y_out = _vq_transform(flat2, codebook)
    return (x_out, y_out)


# speedup vs baseline: 4.2312x; 1.1359x over previous
import functools

import jax
import jax.numpy as jnp
from jax.experimental import pallas as pl
from jax.experimental.pallas import tpu as pltpu

_CODE_DIM = 64
_K = 1024
_SOS_TOKEN = 1024
_N_TOK = 65536
_N_ROWS = _N_TOK // 2
_BLOCK_ROWS = 4096
_BLOCK_TOK = 2 * _BLOCK_ROWS


def _riffle(e, o, n):
    c = jnp.concatenate([e, o], axis=1)
    iota = jax.lax.broadcasted_iota(jnp.int32, (1, 2 * n), 1)
    s = n
    while s >= 2:
        q = s // 2
        b = (iota // q) & 3
        cm = pltpu.roll(c, 2 * n - q, 1)
        cp = pltpu.roll(c, q, 1)
        c = jnp.where(b == 1, cm, jnp.where(b == 2, cp, c))
        s = q
    return c


def _vq_argmin_kernel(cb_ref, x_ref, xout_ref, yout_ref,
                      cbe_scr, cbo_scr, carry_scr):
    @pl.when(pl.program_id(0) == 0)
    def _init():
        cbm2 = -2.0 * cb_ref[...]
        cbsq = 0.25 * jnp.sum(cbm2 * cbm2, axis=1, keepdims=True)
        zero = jnp.zeros_like(cbm2)
        lane64 = jax.lax.broadcasted_iota(jnp.int32, (_K, _CODE_DIM), 1)
        eblk = jnp.where(lane64 == 0, cbsq, 0.0)
        oblk = jnp.where(lane64 == 0, cbsq, zero)
        cbe_scr[...] = jnp.concatenate([cbm2, eblk], axis=1)
        cbo_scr[...] = jnp.concatenate(
            [oblk, cbm2], axis=1)
        carry_scr[0] = _SOS_TOKEN

    x2 = x_ref[...]
    lane = jax.lax.broadcasted_iota(jnp.int32, (_BLOCK_ROWS, 2 * _CODE_DIM), 1)
    x2e = jnp.where(lane == _CODE_DIM, 1.0, x2)
    x2o = jnp.where(lane == 0, 1.0, x2)
    dims = (((1,), (1,)), ((), ()))
    de = jax.lax.dot_general(cbe_scr[...], x2e, dims,
                             preferred_element_type=jnp.float32)
    do = jax.lax.dot_general(cbo_scr[...], x2o, dims,
                             preferred_element_type=jnp.float32)
    idx_e = jnp.argmin(de, axis=0).astype(jnp.int32)
    idx_o = jnp.argmin(do, axis=0).astype(jnp.int32)
    y2 = _riffle(idx_e.reshape(1, _BLOCK_ROWS), idx_o.reshape(1, _BLOCK_ROWS),
                 _BLOCK_ROWS)
    yout_ref[...] = y2.reshape(_BLOCK_TOK)
    rolled = pltpu.roll(y2, 1, 1)
    lt = jax.lax.broadcasted_iota(jnp.int32, (1, _BLOCK_TOK), 1)
    xout_ref[...] = jnp.where(lt == 0, carry_scr[0], rolled).reshape(
        _BLOCK_TOK)
    carry_scr[0] = yout_ref[_BLOCK_TOK - 1]


@functools.partial(jax.jit, static_argnames=("interpret",))
def _vq_transform(flat2, codebook, interpret=False):
    return pl.pallas_call(
        _vq_argmin_kernel,
        grid=(_N_ROWS // _BLOCK_ROWS,),
        in_specs=[
            pl.BlockSpec((_K, _CODE_DIM), lambda i: (0, 0)),
            pl.BlockSpec((_BLOCK_ROWS, 2 * _CODE_DIM), lambda i: (i, 0)),
        ],
        out_specs=[
            pl.BlockSpec((_BLOCK_TOK,), lambda i: (i,)),
            pl.BlockSpec((_BLOCK_TOK,), lambda i: (i,)),
        ],
        out_shape=[
            jax.ShapeDtypeStruct((_N_TOK,), jnp.int32),
            jax.ShapeDtypeStruct((_N_TOK,), jnp.int32),
        ],
        scratch_shapes=[
            pltpu.VMEM((_K, 2 * _CODE_DIM), jnp.float32),
            pltpu.VMEM((_K, 2 * _CODE_DIM), jnp.float32),
            pltpu.SMEM((1,), jnp.int32),
        ],
        interpret=interpret,
    )(codebook, flat2)


def kernel(weights_dict, y, codebook):
    flat2 = weights_dict.reshape(_N_ROWS, 2 * _CODE_DIM)
    x_out, y_out = _vq_transform(flat2, codebook)
    return (x_out, y_out)
